# baseline (speedup 1.0000x reference)
import functools

import jax
import jax.numpy as jnp
from jax.experimental import pallas as pl

N = 10000
F = 128
BM = 400


def _lin_kernel(x_ref, w_ref, b_ref, o_ref):
    o_ref[:] = (
        jnp.dot(x_ref[:], w_ref[:], preferred_element_type=jnp.float32) + b_ref[:]
    )


def _elu(v):
    return jnp.where(v > 0, v, jnp.exp(jnp.minimum(v, 0.0)) - 1.0)


def _split_dot(a, h_hi, h_lo):
    a_hi = a.astype(jnp.bfloat16)
    a_lo = (a - a_hi.astype(jnp.float32)).astype(jnp.bfloat16)
    acc = jnp.dot(a_hi, h_hi, preferred_element_type=jnp.float32)
    acc += jnp.dot(a_lo, h_hi, preferred_element_type=jnp.float32)
    acc += jnp.dot(a_hi, h_lo, preferred_element_type=jnp.float32)
    return acc


def _layer1_kernel(a_ref, h_ref, o_ref):
    h = h_ref[:]
    h_hi = h.astype(jnp.bfloat16)
    h_lo = (h - h_hi.astype(jnp.float32)).astype(jnp.bfloat16)
    acc = _split_dot(a_ref[:], h_hi, h_lo)
    o_ref[:] = _elu(acc)


def _layer2_kernel(a_ref, h_ref, w_ref, b_ref, o_ref):
    h = h_ref[:]
    h_hi = h.astype(jnp.bfloat16)
    h_lo = (h - h_hi.astype(jnp.float32)).astype(jnp.bfloat16)
    acc = _split_dot(a_ref[:], h_hi, h_lo)
    t = _elu(acc)
    o_ref[:] = jnp.dot(t, w_ref[:], preferred_element_type=jnp.float32) + b_ref[:]


@jax.jit
def kernel(x, adjs, W1, b1, Wout, bout):
    W1t = W1.T
    Woutt = Wout.T
    b1r = b1.reshape(1, F)
    boutr = bout.reshape(1, F)

    h0 = pl.pallas_call(
        _lin_kernel,
        out_shape=jax.ShapeDtypeStruct((N, F), jnp.float32),
    )(x, W1t, b1r)

    grid = (N // BM,)
    a_spec = pl.BlockSpec((BM, N), lambda i: (i, 0))
    h_spec = pl.BlockSpec((N, F), lambda i: (0, 0))
    o_spec = pl.BlockSpec((BM, F), lambda i: (i, 0))
    w_spec = pl.BlockSpec((F, F), lambda i: (0, 0))
    b_spec = pl.BlockSpec((1, F), lambda i: (0, 0))

    h1 = pl.pallas_call(
        _layer1_kernel,
        grid=grid,
        in_specs=[a_spec, h_spec],
        out_specs=o_spec,
        out_shape=jax.ShapeDtypeStruct((N, F), jnp.float32),
    )(adjs, h0)

    out = pl.pallas_call(
        _layer2_kernel,
        grid=grid,
        in_specs=[a_spec, h_spec, w_spec, b_spec],
        out_specs=o_spec,
        out_shape=jax.ShapeDtypeStruct((N, F), jnp.float32),
    )(adjs, h1, Woutt, boutr)

    return out


# speedup vs baseline: 1.3683x; 1.3683x over previous
import functools

import jax
import jax.numpy as jnp
from jax.experimental import pallas as pl

N = 10000
F = 128
BM = 400


def _lin_kernel(x_ref, w_ref, b_ref, o_ref):
    o_ref[:] = (
        jnp.dot(x_ref[:], w_ref[:], preferred_element_type=jnp.float32) + b_ref[:]
    )


def _elu(v):
    return jnp.where(v > 0, v, jnp.exp(jnp.minimum(v, 0.0)) - 1.0)


def _layer1_kernel(a_ref, h_ref, o_ref):
    acc = jnp.dot(
        a_ref[:].astype(jnp.bfloat16),
        h_ref[:].astype(jnp.bfloat16),
        preferred_element_type=jnp.float32,
    )
    o_ref[:] = _elu(acc)


def _layer2_kernel(a_ref, h_ref, w_ref, b_ref, o_ref):
    acc = jnp.dot(
        a_ref[:].astype(jnp.bfloat16),
        h_ref[:].astype(jnp.bfloat16),
        preferred_element_type=jnp.float32,
    )
    t = _elu(acc)
    o_ref[:] = jnp.dot(t, w_ref[:], preferred_element_type=jnp.float32) + b_ref[:]


@jax.jit
def kernel(x, adjs, W1, b1, Wout, bout):
    W1t = W1.T
    Woutt = Wout.T
    b1r = b1.reshape(1, F)
    boutr = bout.reshape(1, F)

    h0 = pl.pallas_call(
        _lin_kernel,
        out_shape=jax.ShapeDtypeStruct((N, F), jnp.float32),
    )(x, W1t, b1r)

    grid = (N // BM,)
    a_spec = pl.BlockSpec((BM, N), lambda i: (i, 0))
    h_spec = pl.BlockSpec((N, F), lambda i: (0, 0))
    o_spec = pl.BlockSpec((BM, F), lambda i: (i, 0))
    w_spec = pl.BlockSpec((F, F), lambda i: (0, 0))
    b_spec = pl.BlockSpec((1, F), lambda i: (0, 0))

    h1 = pl.pallas_call(
        _layer1_kernel,
        grid=grid,
        in_specs=[a_spec, h_spec],
        out_specs=o_spec,
        out_shape=jax.ShapeDtypeStruct((N, F), jnp.float32),
    )(adjs, h0)

    out = pl.pallas_call(
        _layer2_kernel,
        grid=grid,
        in_specs=[a_spec, h_spec, w_spec, b_spec],
        out_specs=o_spec,
        out_shape=jax.ShapeDtypeStruct((N, F), jnp.float32),
    )(adjs, h1, Woutt, boutr)

    return out


# speedup vs baseline: 1.4348x; 1.0485x over previous
import jax
import jax.numpy as jnp
from jax.experimental import pallas as pl
from jax.experimental.pallas import tpu as pltpu

N = 10000
F = 128
BM = 400
NB = N // BM


def _elu(v):
    return jnp.where(v > 0, v, jnp.exp(jnp.minimum(v, 0.0)) - 1.0)


def _bf16_dot(a, b):
    return jnp.dot(
        a.astype(jnp.bfloat16),
        b.astype(jnp.bfloat16),
        preferred_element_type=jnp.float32,
    )


def _fused_kernel(
    a_ref, x_ref, w1_ref, b1_ref, wo_ref, bo_ref, o_ref, h0_ref, h1_ref
):
    i = pl.program_id(0)

    @pl.when(i == 0)
    def _():
        h0_ref[:] = _bf16_dot(x_ref[:], w1_ref[:]) + b1_ref[:]

    @pl.when(i < NB)
    def _():
        acc = _bf16_dot(a_ref[:], h0_ref[:])
        h1_ref[pl.ds(i * BM, BM), :] = _elu(acc)

    @pl.when(i >= NB)
    def _():
        acc = _bf16_dot(a_ref[:], h1_ref[:])
        t = _elu(acc)
        o_ref[:] = _bf16_dot(t, wo_ref[:]) + bo_ref[:]


@jax.jit
def kernel(x, adjs, W1, b1, Wout, bout):
    out = pl.pallas_call(
        _fused_kernel,
        grid=(2 * NB,),
        in_specs=[
            pl.BlockSpec((BM, N), lambda i: (i % NB, 0)),
            pl.BlockSpec((N, F), lambda i: (0, 0)),
            pl.BlockSpec((F, F), lambda i: (0, 0)),
            pl.BlockSpec((1, F), lambda i: (0, 0)),
            pl.BlockSpec((F, F), lambda i: (0, 0)),
            pl.BlockSpec((1, F), lambda i: (0, 0)),
        ],
        out_specs=pl.BlockSpec((BM, F), lambda i: (jnp.maximum(i - NB, 0), 0)),
        out_shape=jax.ShapeDtypeStruct((N, F), jnp.float32),
        scratch_shapes=[
            pltpu.VMEM((N, F), jnp.float32),
            pltpu.VMEM((N, F), jnp.float32),
        ],
    )(adjs, x, W1.T, b1.reshape(1, F), Wout.T, bout.reshape(1, F))
    return out


# speedup vs baseline: 1.4372x; 1.0017x over previous
import jax
import jax.numpy as jnp
from jax.experimental import pallas as pl
from jax.experimental.pallas import tpu as pltpu

N = 10000
F = 128
BM = 400
NB = N // BM


def _elu(v):
    return jnp.where(v > 0, v, jnp.exp(jnp.minimum(v, 0.0)) - 1.0)


def _bf16_dot(a, b):
    return jnp.dot(
        a.astype(jnp.bfloat16),
        b.astype(jnp.bfloat16),
        preferred_element_type=jnp.float32,
    )


def _fused_kernel(
    a_ref, x_ref, w1_ref, b1_ref, wo_ref, bo_ref, o_ref, h0_ref, h1_ref
):
    i = pl.program_id(0)

    @pl.when(i == 0)
    def _():
        h0_ref[:] = (_bf16_dot(x_ref[:], w1_ref[:]) + b1_ref[:]).astype(
            jnp.bfloat16
        )

    @pl.when(i < NB)
    def _():
        acc = jnp.dot(
            a_ref[:].astype(jnp.bfloat16),
            h0_ref[:],
            preferred_element_type=jnp.float32,
        )
        h1_ref[pl.ds(i * BM, BM), :] = _elu(acc).astype(jnp.bfloat16)

    @pl.when(i >= NB)
    def _():
        acc = jnp.dot(
            a_ref[:].astype(jnp.bfloat16),
            h1_ref[:],
            preferred_element_type=jnp.float32,
        )
        t = _elu(acc)
        o_ref[:] = _bf16_dot(t, wo_ref[:]) + bo_ref[:]


@jax.jit
def kernel(x, adjs, W1, b1, Wout, bout):
    out = pl.pallas_call(
        _fused_kernel,
        grid=(2 * NB,),
        in_specs=[
            pl.BlockSpec((BM, N), lambda i: (i % NB, 0)),
            pl.BlockSpec((N, F), lambda i: (0, 0)),
            pl.BlockSpec((F, F), lambda i: (0, 0)),
            pl.BlockSpec((1, F), lambda i: (0, 0)),
            pl.BlockSpec((F, F), lambda i: (0, 0)),
            pl.BlockSpec((1, F), lambda i: (0, 0)),
        ],
        out_specs=pl.BlockSpec((BM, F), lambda i: (jnp.maximum(i - NB, 0), 0)),
        out_shape=jax.ShapeDtypeStruct((N, F), jnp.float32),
        scratch_shapes=[
            pltpu.VMEM((N, F), jnp.bfloat16),
            pltpu.VMEM((N, F), jnp.bfloat16),
        ],
    )(adjs, x, W1.T, b1.reshape(1, F), Wout.T, bout.reshape(1, F))
    return out
